# baseline (speedup 1.0000x reference)
import jax
import jax.numpy as jnp
from jax import lax
from jax.experimental import pallas as pl
from jax.experimental.pallas import tpu as pltpu
from jax.experimental.pallas import tpu_sc as plsc

N = 10000
E = 320000
IN_CH = 128
HID = 64
OUT = 2
F = 128

NW = 32
CH = 128
NCHUNK = E // CH
CPW = 78
NXTRA = NCHUNK - NW * CPW
N_PAD = 10240
RPT = N_PAD // 16

_MESH = plsc.VectorSubcoreMesh(core_axis_name="c", subcore_axis_name="s")


def _stage_edges(ei_hbm, idx_v, w):
    pltpu.sync_copy(ei_hbm.at[pl.ds(w * CPW, CPW)], idx_v.at[pl.ds(0, CPW)])

    @pl.when(w < NXTRA)
    def _():
        pltpu.sync_copy(ei_hbm.at[pl.ds(NW * CPW + w, 1)],
                        idx_v.at[pl.ds(CPW, 1)])


def _deg_body(ei_hbm, ones_hbm, z_hbm, out_hbm, idx_v, ones_v, acc, sem):
    c = lax.axis_index("c")
    s = lax.axis_index("s")
    w = c * 16 + s
    pltpu.sync_copy(z_hbm.at[pl.ds(s * RPT, RPT)], acc.at[pl.ds(s * RPT, RPT)])
    pltpu.sync_copy(ones_hbm, ones_v)
    _stage_edges(ei_hbm, idx_v, w)
    plsc.subcore_barrier()

    def group(g, carry):
        for b in range(6):
            pltpu.async_copy(ones_v, acc.at[idx_v.at[g * 6 + b, 1]], sem,
                             add=True)
        for b in range(6):
            pltpu.make_async_copy(ones_v, acc.at[idx_v.at[0, 1]], sem).wait()
        return carry

    lax.fori_loop(0, CPW // 6, group, 0)

    @pl.when(w < NXTRA)
    def _():
        pltpu.sync_copy(ones_v, acc.at[idx_v.at[CPW, 1]], add=True)

    plsc.subcore_barrier()
    pltpu.sync_copy(acc.at[pl.ds(s * RPT, RPT)],
                    out_hbm.at[c, pl.ds(s * RPT, RPT)])


_sc_deg = pl.kernel(
    _deg_body,
    out_type=jax.ShapeDtypeStruct((2, N_PAD), jnp.float32),
    mesh=_MESH,
    compiler_params=pltpu.CompilerParams(use_tc_tiling_on_sc=False, skip_device_barrier=True),
    scratch_types=[
        pltpu.VMEM((CPW + 1, 2, CH), jnp.int32),
        pltpu.VMEM((CH,), jnp.float32),
        pltpu.VMEM_SHARED((N_PAD,), jnp.float32),
        pltpu.SemaphoreType.DMA,
    ],
)


def _make_agg_body(d, mul, scc):
    nsc = CPW // scc

    def body(h_hbm, ei_hbm, z_hbm, out_hbm,
             idx_v, src_v, rows_v, acc, gsemA, gsemB, ssemA, ssemB):
        c = lax.axis_index("c")
        s = lax.axis_index("s")
        w = c * 16 + s
        pltpu.sync_copy(z_hbm.at[pl.ds(s * RPT, RPT)],
                        acc.at[pl.ds(s * RPT, RPT)])
        _stage_edges(ei_hbm, idx_v, w)

        def scale_idx(j, carry):
            for k in range(CH // 16):
                src_v[pl.ds(j * CH + 16 * k, 16)] = (
                    idx_v[j, 0, pl.ds(16 * k, 16)] * mul)
            return carry

        lax.fori_loop(0, CPW + 1, scale_idx, 0)
        plsc.subcore_barrier()

        def fire_gather(g, half, gsem):
            pltpu.async_copy(
                h_hbm.at[src_v.at[pl.ds(g * scc * CH, scc * CH)]],
                rows_v.at[half], gsem)

        def drain_gather(half, gsem):
            pltpu.make_async_copy(h_hbm.at[src_v.at[pl.ds(0, scc * CH)]],
                                  rows_v.at[half], gsem).wait()

        def fire_scatters(g, half, ssem):
            for b in range(scc):
                pltpu.async_copy(rows_v.at[half, pl.ds(b * CH, CH)],
                                 acc.at[idx_v.at[g * scc + b, 1]], ssem,
                                 add=True)

        def drain_scatters(half, ssem):
            for b in range(scc):
                pltpu.make_async_copy(rows_v.at[half, pl.ds(b * CH, CH)],
                                      acc.at[idx_v.at[0, 1]], ssem).wait()

        fire_gather(0, 0, gsemA)

        def step(i, carry):
            gA = 2 * i
            gB = 2 * i + 1
            drain_gather(0, gsemA)
            fire_gather(gB, 1, gsemB)
            fire_scatters(gA, 0, ssemA)
            drain_scatters(0, ssemA)

            @pl.when(gA + 2 < nsc)
            def _():
                fire_gather(gA + 2, 0, gsemA)

            drain_gather(1, gsemB)
            fire_scatters(gB, 1, ssemB)
            drain_scatters(1, ssemB)
            return carry

        lax.fori_loop(0, nsc // 2, step, 0)

        @pl.when(w < NXTRA)
        def _():
            pltpu.async_copy(
                h_hbm.at[src_v.at[pl.ds(CPW * CH, CH)]],
                rows_v.at[0, pl.ds(0, CH)], gsemA).wait()
            pltpu.async_copy(rows_v.at[0, pl.ds(0, CH)],
                             acc.at[idx_v.at[CPW, 1]], ssemA,
                             add=True).wait()

        plsc.subcore_barrier()
        pltpu.sync_copy(acc.at[pl.ds(s * RPT, RPT)],
                        out_hbm.at[c, pl.ds(s * RPT, RPT), pl.ds(0, d)])
    return body


def _make_agg(d, mul, scc):
    return pl.kernel(
        _make_agg_body(d, mul, scc),
        out_type=jax.ShapeDtypeStruct((2, N_PAD, F), jnp.float32),
        mesh=_MESH,
        compiler_params=pltpu.CompilerParams(use_tc_tiling_on_sc=False, skip_device_barrier=True),
        scratch_types=[
            pltpu.VMEM((CPW + 1, 2, CH), jnp.int32),
            pltpu.VMEM(((CPW + 1) * CH,), jnp.int32),
            pltpu.VMEM((2, scc * CH, d), jnp.float32),
            pltpu.VMEM_SHARED((N_PAD, d), jnp.float32),
            pltpu.SemaphoreType.DMA,
            pltpu.SemaphoreType.DMA,
            pltpu.SemaphoreType.DMA,
            pltpu.SemaphoreType.DMA,
        ],
    )


_sc_agg64 = _make_agg(HID, F // HID, 3)
_sc_agg16 = _make_agg(16, F // 16, 13)


def _dinv_from(degp):
    deg = degp[0] + degp[1] + 1.0
    return lax.rsqrt(deg).reshape(N_PAD, 1)


def _prep_body(x_ref, w1_ref, degp_ref, h1s_ref):
    dinv = _dinv_from(degp_ref[...])
    m1 = jnp.dot(x_ref[...], w1_ref[...], preferred_element_type=jnp.float32)
    h1s_ref[pl.ds(0, N), pl.ds(0, HID)] = m1 * dinv[:N, :]


_tc_prep = pl.pallas_call(
    _prep_body,
    out_shape=jax.ShapeDtypeStruct((N_PAD, F), jnp.float32),
)


def _mid_body(u1p_ref, h1s_ref, degp_ref, b1_ref, w2_ref, g2s_ref):
    dinv = _dinv_from(degp_ref[...])
    u1 = u1p_ref[0, :, :HID] + u1p_ref[1, :, :HID] + h1s_ref[:, :HID]
    out1 = u1 * dinv + b1_ref[...].reshape(1, HID)
    h2 = jnp.maximum(out1, 0.0)
    g2 = jnp.dot(h2, w2_ref[...], preferred_element_type=jnp.float32)
    g2s_ref[:, pl.ds(0, 16)] = g2 * dinv


_tc_mid = pl.pallas_call(
    _mid_body,
    out_shape=jax.ShapeDtypeStruct((N_PAD, F), jnp.float32),
)


def _fin_body(u2p_ref, g2s_ref, degp_ref, b2_ref, out_ref):
    dinv = _dinv_from(degp_ref[...])
    u2 = u2p_ref[0, :, :16] + u2p_ref[1, :, :16] + g2s_ref[:, :16]
    res = u2 * dinv + b2_ref[...].reshape(1, 16)
    out_ref[...] = res[:N, :OUT]


_tc_fin = pl.pallas_call(
    _fin_body,
    out_shape=jax.ShapeDtypeStruct((N, OUT), jnp.float32),
)


def kernel(x, edge_index, W1, b1, W2, b2):
    ei3 = jnp.transpose(edge_index.reshape(2, NCHUNK, CH), (1, 0, 2))

    ones_ch = jnp.ones((CH,), jnp.float32)
    z1 = jnp.zeros((N_PAD,), jnp.float32)
    z64 = jnp.zeros((N_PAD, HID), jnp.float32)
    z16 = jnp.zeros((N_PAD, 16), jnp.float32)
    w2_p = jnp.pad(W2, ((0, 0), (0, 16 - OUT)))
    b2_p = jnp.pad(b2, ((0, 16 - OUT),))

    deg_p = _sc_deg(ei3, ones_ch, z1)
    h1s = _tc_prep(x, W1, deg_p)
    u1_p = _sc_agg64(h1s.reshape(N_PAD * 2, HID), ei3, z64)
    g2s = _tc_mid(u1_p, h1s, deg_p, b1, w2_p)
    u2_p = _sc_agg16(g2s.reshape(N_PAD * 8, 16), ei3, z16)
    return _tc_fin(u2_p, g2s, deg_p, b2_p)


# speedup vs baseline: 1.0037x; 1.0037x over previous
import jax
import jax.numpy as jnp
from jax import lax
from jax.experimental import pallas as pl
from jax.experimental.pallas import tpu as pltpu
from jax.experimental.pallas import tpu_sc as plsc

N = 10000
E = 320000
IN_CH = 128
HID = 64
OUT = 2
F = 128

NW = 32
CH = 128
NCHUNK = E // CH
CPW = 78
NXTRA = NCHUNK - NW * CPW
N_PAD = 10240
RPT = N_PAD // 16

_MESH = plsc.VectorSubcoreMesh(core_axis_name="c", subcore_axis_name="s")


def _stage_edges(ei_hbm, idx_v, w):
    pltpu.sync_copy(ei_hbm.at[pl.ds(w * CPW, CPW)], idx_v.at[pl.ds(0, CPW)])

    @pl.when(w < NXTRA)
    def _():
        pltpu.sync_copy(ei_hbm.at[pl.ds(NW * CPW + w, 1)],
                        idx_v.at[pl.ds(CPW, 1)])


def _deg_body(ei_hbm, ones_hbm, z_hbm, out_hbm, idx_v, ones_v, acc, sem):
    c = lax.axis_index("c")
    s = lax.axis_index("s")
    w = c * 16 + s
    pltpu.sync_copy(z_hbm.at[pl.ds(s * RPT, RPT)], acc.at[pl.ds(s * RPT, RPT)])
    pltpu.sync_copy(ones_hbm, ones_v)
    _stage_edges(ei_hbm, idx_v, w)
    plsc.subcore_barrier()

    def group(g, carry):
        for b in range(13):
            pltpu.async_copy(ones_v, acc.at[idx_v.at[g * 13 + b, 1]], sem,
                             add=True)
        for b in range(13):
            pltpu.make_async_copy(ones_v, acc.at[idx_v.at[0, 1]], sem).wait()
        return carry

    lax.fori_loop(0, CPW // 13, group, 0)

    @pl.when(w < NXTRA)
    def _():
        pltpu.sync_copy(ones_v, acc.at[idx_v.at[CPW, 1]], add=True)

    plsc.subcore_barrier()
    pltpu.sync_copy(acc.at[pl.ds(s * RPT, RPT)],
                    out_hbm.at[c, pl.ds(s * RPT, RPT)])


_sc_deg = pl.kernel(
    _deg_body,
    out_type=jax.ShapeDtypeStruct((2, N_PAD), jnp.float32),
    mesh=_MESH,
    compiler_params=pltpu.CompilerParams(use_tc_tiling_on_sc=False),
    scratch_types=[
        pltpu.VMEM((CPW + 1, 2, CH), jnp.int32),
        pltpu.VMEM((CH,), jnp.float32),
        pltpu.VMEM_SHARED((N_PAD,), jnp.float32),
        pltpu.SemaphoreType.DMA,
    ],
)


def _make_agg_body(d, mul, scc):
    nsc = CPW // scc

    def body(h_hbm, ei_hbm, z_hbm, out_hbm,
             idx_v, src_v, rows_v, acc, gsemA, gsemB, ssemA, ssemB):
        c = lax.axis_index("c")
        s = lax.axis_index("s")
        w = c * 16 + s
        pltpu.sync_copy(z_hbm.at[pl.ds(s * RPT, RPT)],
                        acc.at[pl.ds(s * RPT, RPT)])
        _stage_edges(ei_hbm, idx_v, w)

        def scale_idx(j, carry):
            for k in range(CH // 16):
                src_v[pl.ds(j * CH + 16 * k, 16)] = (
                    idx_v[j, 0, pl.ds(16 * k, 16)] * mul)
            return carry

        lax.fori_loop(0, CPW + 1, scale_idx, 0)
        plsc.subcore_barrier()

        def fire_gather(g, half, gsem):
            pltpu.async_copy(
                h_hbm.at[src_v.at[pl.ds(g * scc * CH, scc * CH)]],
                rows_v.at[half], gsem)

        def drain_gather(half, gsem):
            pltpu.make_async_copy(h_hbm.at[src_v.at[pl.ds(0, scc * CH)]],
                                  rows_v.at[half], gsem).wait()

        def fire_scatters(g, half, ssem):
            for b in range(scc):
                pltpu.async_copy(rows_v.at[half, pl.ds(b * CH, CH)],
                                 acc.at[idx_v.at[g * scc + b, 1]], ssem,
                                 add=True)

        def drain_scatters(half, ssem):
            for b in range(scc):
                pltpu.make_async_copy(rows_v.at[half, pl.ds(b * CH, CH)],
                                      acc.at[idx_v.at[0, 1]], ssem).wait()

        fire_gather(0, 0, gsemA)

        def step(i, carry):
            gA = 2 * i
            gB = 2 * i + 1
            drain_gather(0, gsemA)
            fire_gather(gB, 1, gsemB)
            fire_scatters(gA, 0, ssemA)
            drain_scatters(0, ssemA)

            @pl.when(gA + 2 < nsc)
            def _():
                fire_gather(gA + 2, 0, gsemA)

            drain_gather(1, gsemB)
            fire_scatters(gB, 1, ssemB)
            drain_scatters(1, ssemB)
            return carry

        lax.fori_loop(0, nsc // 2, step, 0)

        @pl.when(w < NXTRA)
        def _():
            pltpu.async_copy(
                h_hbm.at[src_v.at[pl.ds(CPW * CH, CH)]],
                rows_v.at[0, pl.ds(0, CH)], gsemA).wait()
            pltpu.async_copy(rows_v.at[0, pl.ds(0, CH)],
                             acc.at[idx_v.at[CPW, 1]], ssemA,
                             add=True).wait()

        plsc.subcore_barrier()
        pltpu.sync_copy(acc.at[pl.ds(s * RPT, RPT)],
                        out_hbm.at[c, pl.ds(s * RPT, RPT), pl.ds(0, d)])
    return body


def _make_agg(d, mul, scc):
    return pl.kernel(
        _make_agg_body(d, mul, scc),
        out_type=jax.ShapeDtypeStruct((2, N_PAD, F), jnp.float32),
        mesh=_MESH,
        compiler_params=pltpu.CompilerParams(use_tc_tiling_on_sc=False),
        scratch_types=[
            pltpu.VMEM((CPW + 1, 2, CH), jnp.int32),
            pltpu.VMEM(((CPW + 1) * CH,), jnp.int32),
            pltpu.VMEM((2, scc * CH, d), jnp.float32),
            pltpu.VMEM_SHARED((N_PAD, d), jnp.float32),
            pltpu.SemaphoreType.DMA,
            pltpu.SemaphoreType.DMA,
            pltpu.SemaphoreType.DMA,
            pltpu.SemaphoreType.DMA,
        ],
    )


_sc_agg64 = _make_agg(HID, F // HID, 3)
_sc_agg16 = _make_agg(16, F // 16, 13)


def _dinv_from(degp):
    deg = degp[0] + degp[1] + 1.0
    return lax.rsqrt(deg).reshape(N_PAD, 1)


def _prep_body(x_ref, w1_ref, degp_ref, h1s_ref):
    dinv = _dinv_from(degp_ref[...])
    m1 = jnp.dot(x_ref[...], w1_ref[...], preferred_element_type=jnp.float32)
    h1s_ref[pl.ds(0, N), pl.ds(0, HID)] = m1 * dinv[:N, :]


_tc_prep = pl.pallas_call(
    _prep_body,
    out_shape=jax.ShapeDtypeStruct((N_PAD, F), jnp.float32),
)


def _mid_body(u1p_ref, h1s_ref, degp_ref, b1_ref, w2_ref, g2s_ref):
    dinv = _dinv_from(degp_ref[...])
    u1 = u1p_ref[0, :, :HID] + u1p_ref[1, :, :HID] + h1s_ref[:, :HID]
    out1 = u1 * dinv + b1_ref[...].reshape(1, HID)
    h2 = jnp.maximum(out1, 0.0)
    g2 = jnp.dot(h2, w2_ref[...], preferred_element_type=jnp.float32)
    g2s_ref[:, pl.ds(0, 16)] = g2 * dinv


_tc_mid = pl.pallas_call(
    _mid_body,
    out_shape=jax.ShapeDtypeStruct((N_PAD, F), jnp.float32),
)


def _fin_body(u2p_ref, g2s_ref, degp_ref, b2_ref, out_ref):
    dinv = _dinv_from(degp_ref[...])
    u2 = u2p_ref[0, :, :16] + u2p_ref[1, :, :16] + g2s_ref[:, :16]
    res = u2 * dinv + b2_ref[...].reshape(1, 16)
    out_ref[...] = res[:N, :OUT]


_tc_fin = pl.pallas_call(
    _fin_body,
    out_shape=jax.ShapeDtypeStruct((N, OUT), jnp.float32),
)


def kernel(x, edge_index, W1, b1, W2, b2):
    ei3 = jnp.transpose(edge_index.reshape(2, NCHUNK, CH), (1, 0, 2))

    ones_ch = jnp.ones((CH,), jnp.float32)
    z1 = jnp.zeros((N_PAD,), jnp.float32)
    z64 = jnp.zeros((N_PAD, HID), jnp.float32)
    z16 = jnp.zeros((N_PAD, 16), jnp.float32)
    w2_p = jnp.pad(W2, ((0, 0), (0, 16 - OUT)))
    b2_p = jnp.pad(b2, ((0, 16 - OUT),))

    deg_p = _sc_deg(ei3, ones_ch, z1)
    h1s = _tc_prep(x, W1, deg_p)
    u1_p = _sc_agg64(h1s.reshape(N_PAD * 2, HID), ei3, z64)
    g2s = _tc_mid(u1_p, h1s, deg_p, b1, w2_p)
    u2_p = _sc_agg16(g2s.reshape(N_PAD * 8, 16), ei3, z16)
    return _tc_fin(u2_p, g2s, deg_p, b2_p)
